# baseline (speedup 1.0000x reference)
import functools

import jax
import jax.numpy as jnp
from jax import lax
from jax.experimental import pallas as pl
from jax.experimental.pallas import tpu as pltpu
from jax.experimental.pallas import tpu_sc as plsc


def kernel(token_ids, weight):
    V, D = weight.shape
    B, S = token_ids.shape
    NB = B // 128
    SP = 56

    wT = weight.T
    idTp = jnp.pad(token_ids.T, ((0, SP - S), (0, 0)))

    mesh = plsc.VectorSubcoreMesh(core_axis_name="c", subcore_axis_name="s")

    NFULL = V // 128

    @functools.partial(
        pl.kernel,
        mesh=mesh,
        compiler_params=pltpu.CompilerParams(needs_layout_passes=False),
        out_type=jax.ShapeDtypeStruct((V // 2, 128), jnp.float32),
        scratch_types=[
            pltpu.VMEM((2, 64, 128), jnp.float32),
            pltpu.VMEM((64, 128), jnp.float32),
            pltpu.VMEM((64, 64), jnp.float32),
            pltpu.SemaphoreType.DMA((2,)),
        ],
    )
    def k1(wt_hbm, w2_hbm, slab_v, blk_v, tail_v, sems):
        wid = lax.axis_index("s") * 2 + lax.axis_index("c")
        rows_g = [lax.iota(jnp.int32, 16) + 16 * g for g in range(4)]

        def start(k, slot):
            j = wid + 32 * k
            c0 = pl.multiple_of(j * 128, 128)
            pltpu.async_copy(
                wt_hbm.at[:, pl.ds(c0, 128)], slab_v.at[slot], sems.at[slot]
            )

        def wait(slot):
            pltpu.make_async_copy(
                wt_hbm.at[:, pl.ds(0, 128)], slab_v.at[slot], sems.at[slot]
            ).wait()

        def transpose_store(k, slot):
            j = wid + 32 * k
            for p in range(64):
                for h in range(2):
                    col = jnp.full((16,), 2 * p + h, jnp.int32)
                    for g in range(4):
                        val = plsc.load_gather(
                            slab_v.at[slot], [rows_g[g], col]
                        )
                        blk_v[p, pl.ds(h * 64 + g * 16, 16)] = val
            r0 = pl.multiple_of(j * 64, 64)
            pltpu.sync_copy(blk_v, w2_hbm.at[pl.ds(r0, 64)])

        start(0, 0)
        start(1, 1)

        def body(r, _):
            for s2 in range(2):
                k = 2 * r + s2
                wait(s2)
                transpose_store(k, s2)
                start(k + 2, s2)
            return 0

        lax.fori_loop(0, 121, body, 0)
        for s2 in range(2):
            wait(s2)
            transpose_store(242 + s2, s2)

        @pl.when(wid < 4)
        def _():
            start(244, 0)
            wait(0)
            transpose_store(244, 0)

        @pl.when(wid == 4)
        def _():
            pltpu.sync_copy(wt_hbm.at[:, pl.ds(NFULL * 128, 64)], tail_v)
            for p in range(32):
                for h in range(2):
                    col = jnp.full((16,), 2 * p + h, jnp.int32)
                    for g in range(4):
                        val = plsc.load_gather(tail_v, [rows_g[g], col])
                        blk_v[p, pl.ds(h * 64 + g * 16, 16)] = val
            pltpu.sync_copy(
                blk_v.at[pl.ds(0, 32)], w2_hbm.at[pl.ds(NFULL * 64, 32)]
            )

    w2 = k1(wT)

    @functools.partial(
        pl.kernel,
        mesh=mesh,
        compiler_params=pltpu.CompilerParams(needs_layout_passes=False),
        out_type=jax.ShapeDtypeStruct((S, 8, NB, 8, 128), jnp.float32),
        scratch_types=[
            pltpu.VMEM((SP, 128), jnp.int32),
            pltpu.VMEM((2, 128), jnp.int32),
            pltpu.VMEM((2, 128, 128), jnp.float32),
            pltpu.VMEM((8, 8, 128), jnp.float32),
            pltpu.SemaphoreType.DMA((2,)),
        ],
    )
    def k2(idt_hbm, w2_hbm, o_hbm, idx_v, gidx_v, rows_v, blk_v, sems):
        wid = lax.axis_index("s") * 2 + lax.axis_index("c")
        c0 = pl.multiple_of(wid * 128, 128)
        pltpu.sync_copy(idt_hbm.at[:, pl.ds(c0, 128)], idx_v)
        rows_g = [lax.iota(jnp.int32, 16) + 16 * g for g in range(8)]

        def prep_start(s, slot):
            for g in range(8):
                idv = idx_v[s, pl.ds(16 * g, 16)]
                gidx_v[slot, pl.ds(16 * g, 16)] = lax.shift_right_logical(
                    idv, 1
                )
            pltpu.async_copy(
                w2_hbm.at[gidx_v.at[slot]], rows_v.at[slot], sems.at[slot]
            )

        def wait(slot):
            pltpu.make_async_copy(
                w2_hbm.at[gidx_v.at[slot]], rows_v.at[slot], sems.at[slot]
            ).wait()

        def transpose_store(s, slot):
            hb = []
            for g in range(8):
                idv = idx_v[s, pl.ds(16 * g, 16)]
                hb.append(lax.shift_left(lax.bitwise_and(idv, 1), 6))
            for dt in range(8):
                for dr in range(8):
                    d = dt * 8 + dr
                    for g in range(8):
                        col = hb[g] + d
                        val = plsc.load_gather(
                            rows_v.at[slot], [rows_g[g], col]
                        )
                        blk_v[dt, dr, pl.ds(16 * g, 16)] = val
            for dt in range(8):
                pltpu.sync_copy(blk_v.at[dt], o_hbm.at[s, dt, wid])

        prep_start(0, 0)
        prep_start(1, 1)

        def body(r, _):
            for s2 in range(2):
                s = 2 * r + s2
                wait(s2)
                transpose_store(s, s2)
                prep_start(s + 2, s2)
            return 0

        lax.fori_loop(0, (S - 2) // 2, body, 0)
        for s2 in range(2):
            wait(s2)
            transpose_store(48 + s2, s2)

    o = k2(idTp, w2)
    return jnp.transpose(o, (2, 4, 0, 1, 3)).reshape(B, S, D)


# speedup vs baseline: 1.6000x; 1.6000x over previous
import functools

import jax
import jax.numpy as jnp
from jax import lax
from jax.experimental import pallas as pl
from jax.experimental.pallas import tpu as pltpu
from jax.experimental.pallas import tpu_sc as plsc

NBUF = 8


def kernel(token_ids, weight):
    B, S = token_ids.shape
    V, D = weight.shape
    SP = 56

    info = plsc.get_sparse_core_info()
    NW = info.num_cores * info.num_subcores
    BPW = B // NW

    idx3d = jnp.pad(token_ids, ((0, 0), (0, SP - S))).reshape(NW, BPW, SP)

    mesh = plsc.VectorSubcoreMesh(core_axis_name="c", subcore_axis_name="s")

    @functools.partial(
        pl.kernel,
        mesh=mesh,
        compiler_params=pltpu.CompilerParams(use_tc_tiling_on_sc=False),
        out_type=jax.ShapeDtypeStruct((B, S, D), jnp.float32),
        scratch_types=[
            pltpu.VMEM((BPW, SP), jnp.int32),
            pltpu.VMEM((NBUF, SP, D), jnp.float32),
            pltpu.SemaphoreType.DMA((NBUF,)),
        ],
    )
    def k(idx_hbm, table_hbm, out_hbm, idx_v, rows_v, sems):
        wid = lax.axis_index("s") * info.num_cores + lax.axis_index("c")
        b0 = wid * BPW
        pltpu.sync_copy(idx_hbm.at[wid], idx_v)

        def start(b, slot):
            pltpu.async_copy(
                table_hbm.at[idx_v.at[b]], rows_v.at[slot], sems.at[slot]
            )

        def wait(slot):
            pltpu.make_async_copy(
                table_hbm.at[idx_v.at[0]], rows_v.at[slot], sems.at[slot]
            ).wait()

        def store(b, slot):
            pltpu.sync_copy(
                rows_v.at[slot, pl.ds(0, S)], out_hbm.at[b0 + b]
            )

        for slot in range(NBUF):
            start(slot, slot)

        def body(r, _):
            for slot in range(NBUF):
                b = r * NBUF + slot
                wait(slot)
                store(b, slot)
                start(b + NBUF, slot)
            return 0

        lax.fori_loop(0, BPW // NBUF - 1, body, 0)
        for slot in range(NBUF):
            b = BPW - NBUF + slot
            wait(slot)
            store(b, slot)

    return k(idx3d, weight)


# speedup vs baseline: 1.6024x; 1.0015x over previous
import functools

import jax
import jax.numpy as jnp
from jax import lax
from jax.experimental import pallas as pl
from jax.experimental.pallas import tpu as pltpu
from jax.experimental.pallas import tpu_sc as plsc

NBUF = 8


def kernel(token_ids, weight):
    B, S = token_ids.shape
    V, D = weight.shape
    SP = 56
    CW = 2 * SP

    info = plsc.get_sparse_core_info()
    NW = info.num_cores * info.num_subcores
    BPW = B // NW
    NCH = BPW // 2

    idx3d = jnp.pad(token_ids, ((0, 0), (0, SP - S))).reshape(NW, NCH, CW)

    mesh = plsc.VectorSubcoreMesh(core_axis_name="c", subcore_axis_name="s")

    @functools.partial(
        pl.kernel,
        mesh=mesh,
        compiler_params=pltpu.CompilerParams(use_tc_tiling_on_sc=False),
        out_type=jax.ShapeDtypeStruct((B, S, D), jnp.float32),
        scratch_types=[
            pltpu.VMEM((NCH, CW), jnp.int32),
            pltpu.VMEM((NBUF, CW, D), jnp.float32),
            pltpu.SemaphoreType.DMA((NBUF,)),
        ],
    )
    def k(idx_hbm, table_hbm, out_hbm, idx_v, rows_v, sems):
        wid = lax.axis_index("s") * info.num_cores + lax.axis_index("c")
        b0 = wid * BPW
        pltpu.sync_copy(idx_hbm.at[wid], idx_v)

        def start(c, slot):
            pltpu.async_copy(
                table_hbm.at[idx_v.at[c]], rows_v.at[slot], sems.at[slot]
            )

        def wait(slot):
            pltpu.make_async_copy(
                table_hbm.at[idx_v.at[0]], rows_v.at[slot], sems.at[slot]
            ).wait()

        def store(c, slot):
            bg = b0 + 2 * c
            pltpu.sync_copy(rows_v.at[slot, pl.ds(0, S)], out_hbm.at[bg])
            pltpu.sync_copy(rows_v.at[slot, pl.ds(SP, S)], out_hbm.at[bg + 1])

        for slot in range(NBUF):
            start(slot, slot)

        def body(r, _):
            for slot in range(NBUF):
                c = r * NBUF + slot
                wait(slot)
                store(c, slot)
                start(c + NBUF, slot)
            return 0

        lax.fori_loop(0, NCH // NBUF - 1, body, 0)
        for slot in range(NBUF):
            c = NCH - NBUF + slot
            wait(slot)
            store(c, slot)

    return k(idx3d, weight)


# speedup vs baseline: 2.7111x; 1.6919x over previous
import functools

import jax
import jax.numpy as jnp
from jax import lax
from jax.experimental import pallas as pl
from jax.experimental.pallas import tpu as pltpu
from jax.experimental.pallas import tpu_sc as plsc

NBUF = 8


def kernel(token_ids, weight):
    B, S = token_ids.shape
    V, D = weight.shape
    SP = 56
    CW = 2 * SP

    info = plsc.get_sparse_core_info()
    NW = info.num_cores * info.num_subcores
    BPW = B // NW
    NCH = BPW // 2

    pad_ids = (jnp.arange(B * (SP - S), dtype=token_ids.dtype) * 4099) % V
    idx3d = jnp.concatenate(
        [token_ids, pad_ids.reshape(B, SP - S)], axis=1
    ).reshape(NW, NCH, CW)

    mesh = plsc.VectorSubcoreMesh(core_axis_name="c", subcore_axis_name="s")

    @functools.partial(
        pl.kernel,
        mesh=mesh,
        compiler_params=pltpu.CompilerParams(use_tc_tiling_on_sc=False),
        out_type=jax.ShapeDtypeStruct((B, S, D), jnp.float32),
        scratch_types=[
            pltpu.VMEM((NCH, CW), jnp.int32),
            pltpu.VMEM((NBUF, CW, D), jnp.float32),
            pltpu.SemaphoreType.DMA((NBUF,)),
        ],
    )
    def k(idx_hbm, table_hbm, out_hbm, idx_v, rows_v, sems):
        wid = lax.axis_index("s") * info.num_cores + lax.axis_index("c")
        b0 = wid * BPW
        pltpu.sync_copy(idx_hbm.at[wid], idx_v)

        def start(c, slot):
            pltpu.async_copy(
                table_hbm.at[idx_v.at[c]], rows_v.at[slot], sems.at[slot]
            )

        def wait(slot):
            pltpu.make_async_copy(
                table_hbm.at[idx_v.at[0]], rows_v.at[slot], sems.at[slot]
            ).wait()

        def store(c, slot):
            bg = b0 + 2 * c
            pltpu.sync_copy(rows_v.at[slot, pl.ds(0, S)], out_hbm.at[bg])
            pltpu.sync_copy(rows_v.at[slot, pl.ds(SP, S)], out_hbm.at[bg + 1])

        for slot in range(NBUF):
            start(slot, slot)

        def body(r, _):
            for slot in range(NBUF):
                c = r * NBUF + slot
                wait(slot)
                store(c, slot)
                start(c + NBUF, slot)
            return 0

        lax.fori_loop(0, NCH // NBUF - 1, body, 0)
        for slot in range(NBUF):
            c = NCH - NBUF + slot
            wait(slot)
            store(c, slot)

    return k(idx3d, weight)
